# baseline (speedup 1.0000x reference)
import functools

import jax
import jax.numpy as jnp
from jax import lax
from jax.experimental import pallas as pl
from jax.experimental.pallas import tpu as pltpu
from jax.experimental.pallas import tpu_sc as plsc

B, L = 4096, 200
CHAR_D, SEG_D, POS_D, BICHAR_D = 64, 16, 32, 64
D_TOT = CHAR_D + SEG_D + POS_D + BICHAR_D
N = B * L

NC, NS = 2, 16
NW = NC * NS
TOK_PER_W = N // NW
CHUNK = 128
SUP = 2
STAGE_TOK = SUP * CHUNK
NSTAGE = TOK_PER_W // STAGE_TOK

_mesh = plsc.VectorSubcoreMesh(core_axis_name="c", subcore_axis_name="s")

_COLS = (
    (0, 0, CHAR_D),
    (1, CHAR_D, SEG_D),
    (2, CHAR_D + SEG_D, POS_D),
    (3, CHAR_D + SEG_D + POS_D, BICHAR_D),
)


@functools.partial(
    pl.kernel,
    mesh=_mesh,
    out_type=jax.ShapeDtypeStruct((N, D_TOT), jnp.float32),
    scratch_types=[
        pltpu.VMEM((SUP * 4, CHUNK), jnp.int32),
        pltpu.VMEM((SUP * 4, CHUNK), jnp.int32),
        [pltpu.VMEM((STAGE_TOK, d), jnp.float32) for d in (CHAR_D, SEG_D, POS_D, BICHAR_D)],
        [pltpu.VMEM((STAGE_TOK, d), jnp.float32) for d in (CHAR_D, SEG_D, POS_D, BICHAR_D)],
        pltpu.SemaphoreType.DMA,
        pltpu.SemaphoreType.DMA,
        pltpu.SemaphoreType.DMA,
        pltpu.SemaphoreType.DMA,
        pltpu.SemaphoreType.DMA,
        pltpu.SemaphoreType.DMA,
    ],
    compiler_params=pltpu.CompilerParams(use_tc_tiling_on_sc=False),
)
def _mix_embed(idx_hbm, char_hbm, seg_hbm, pos_hbm, bichar_hbm, out_hbm,
               idx0, idx1, bufs0, bufs1, is0, is1, gs0, gs1, ws0, ws1):
    idx_v = (idx0, idx1)
    tab_v = (bufs0, bufs1)
    isem = (is0, is1)
    gsem = (gs0, gs1)
    wsem = (ws0, ws1)
    tables = (char_hbm, seg_hbm, pos_hbm, bichar_hbm)
    wid = lax.axis_index("s") * NC + lax.axis_index("c")
    first_stage = wid * NSTAGE

    def fire_idx(s, b):
        return pltpu.async_copy(idx_hbm.at[first_stage + s], idx_v[b], isem[b])

    def gather_copies(b, fn):
        out = []
        for c in range(SUP):
            tok = c * CHUNK
            for t, (row, col, width) in enumerate(_COLS):
                out.append(fn(
                    tables[t].at[idx_v[b].at[c * 4 + row]],
                    tab_v[b][t].at[pl.ds(tok, CHUNK)],
                    gsem[b],
                ))
        return out

    def write_copies(s, b, fn):
        base = (first_stage + s) * STAGE_TOK
        return [
            fn(tab_v[b][t], out_hbm.at[pl.ds(base, STAGE_TOK), pl.ds(col, width)], wsem[b])
            for t, (row, col, width) in enumerate(_COLS)
        ]

    fire_idx(0, 0)

    def body(ss, carry):
        for b in (0, 1):
            s = ss * 2 + b
            nb = 1 - b
            pltpu.make_async_copy(idx_hbm.at[first_stage + s], idx_v[b], isem[b]).wait()
            @pl.when(s >= 2)
            def _():
                for w in write_copies(s - 2, b, pltpu.make_async_copy):
                    w.wait()
            gather_copies(b, pltpu.async_copy)
            @pl.when(s + 1 < NSTAGE)
            def _():
                fire_idx(s + 1, nb)
            for w in gather_copies(b, pltpu.make_async_copy):
                w.wait()
            write_copies(s, b, pltpu.async_copy)
        return carry

    lax.fori_loop(0, NSTAGE // 2, body, 0)
    for w in write_copies(NSTAGE - 2, 0, pltpu.make_async_copy):
        w.wait()
    for w in write_copies(NSTAGE - 1, 1, pltpu.make_async_copy):
        w.wait()


def kernel(pad_chars, pad_bichars, pad_segs, pad_poss, char_table, bichar_table, seg_table, pos_table):
    idx = jnp.stack(
        [pad_chars.reshape(-1), pad_segs.reshape(-1), pad_poss.reshape(-1), pad_bichars.reshape(-1)],
        axis=0,
    ).astype(jnp.int32)
    idx = (idx.reshape(4, N // CHUNK, CHUNK)
              .transpose(1, 0, 2)
              .reshape(N // STAGE_TOK, SUP * 4, CHUNK))
    out = _mix_embed(idx, char_table, seg_table, pos_table, bichar_table)
    return out.reshape(B, L, D_TOT)


# speedup vs baseline: 1.0081x; 1.0081x over previous
import functools

import jax
import jax.numpy as jnp
from jax import lax
from jax.experimental import pallas as pl
from jax.experimental.pallas import tpu as pltpu
from jax.experimental.pallas import tpu_sc as plsc

B, L = 4096, 200
CHAR_D, SEG_D, POS_D, BICHAR_D = 64, 16, 32, 64
D_TOT = CHAR_D + SEG_D + POS_D + BICHAR_D
N = B * L

NC, NS = 2, 16
NW = NC * NS
TOK_PER_W = N // NW
CHUNK = 128
SUP = 2
STAGE_TOK = SUP * CHUNK
NSTAGE = TOK_PER_W // STAGE_TOK

_mesh = plsc.VectorSubcoreMesh(core_axis_name="c", subcore_axis_name="s")

_COLS = (
    (0, CHAR_D),
    (CHAR_D, SEG_D),
    (CHAR_D + SEG_D, POS_D),
    (CHAR_D + SEG_D + POS_D, BICHAR_D),
)
_DIMS = (CHAR_D, SEG_D, POS_D, BICHAR_D)


@functools.partial(
    pl.kernel,
    mesh=_mesh,
    out_type=jax.ShapeDtypeStruct((N, D_TOT), jnp.float32),
    scratch_types=[
        [pltpu.VMEM((SUP, CHUNK), jnp.int32) for _ in range(4)],
        [pltpu.VMEM((SUP, CHUNK), jnp.int32) for _ in range(4)],
        [pltpu.VMEM((STAGE_TOK, d), jnp.float32) for d in _DIMS],
        [pltpu.VMEM((STAGE_TOK, d), jnp.float32) for d in _DIMS],
        pltpu.SemaphoreType.DMA,
        pltpu.SemaphoreType.DMA,
        pltpu.SemaphoreType.DMA,
        pltpu.SemaphoreType.DMA,
        pltpu.SemaphoreType.DMA,
        pltpu.SemaphoreType.DMA,
    ],
    compiler_params=pltpu.CompilerParams(use_tc_tiling_on_sc=False),
)
def _mix_embed(ic_hbm, is_hbm, ip_hbm, ib_hbm,
               char_hbm, seg_hbm, pos_hbm, bichar_hbm, out_hbm,
               idx0, idx1, bufs0, bufs1, is0, is1, gs0, gs1, ws0, ws1):
    idx_v = (idx0, idx1)
    tab_v = (bufs0, bufs1)
    isem = (is0, is1)
    gsem = (gs0, gs1)
    wsem = (ws0, ws1)
    idx_hbm = (ic_hbm, is_hbm, ip_hbm, ib_hbm)
    tables = (char_hbm, seg_hbm, pos_hbm, bichar_hbm)
    wid = lax.axis_index("s") * NC + lax.axis_index("c")
    first_stage = wid * NSTAGE

    def idx_copies(s, b, fn):
        row = (first_stage + s) * SUP
        return [fn(idx_hbm[t].at[pl.ds(row, SUP)], idx_v[b][t], isem[b])
                for t in range(4)]

    def gather_copies(b, fn):
        out = []
        for c in range(SUP):
            tok = c * CHUNK
            for t in range(4):
                out.append(fn(
                    tables[t].at[idx_v[b][t].at[c]],
                    tab_v[b][t].at[pl.ds(tok, CHUNK)],
                    gsem[b],
                ))
        return out

    def write_copies(s, b, fn):
        base = (first_stage + s) * STAGE_TOK
        return [
            fn(tab_v[b][t], out_hbm.at[pl.ds(base, STAGE_TOK), pl.ds(col, width)], wsem[b])
            for t, (col, width) in enumerate(_COLS)
        ]

    idx_copies(0, 0, pltpu.async_copy)

    def body(ss, carry):
        for b in (0, 1):
            s = ss * 2 + b
            nb = 1 - b
            for w in idx_copies(s, b, pltpu.make_async_copy):
                w.wait()
            @pl.when(s >= 2)
            def _():
                for w in write_copies(s - 2, b, pltpu.make_async_copy):
                    w.wait()
            gather_copies(b, pltpu.async_copy)
            @pl.when(s + 1 < NSTAGE)
            def _():
                idx_copies(s + 1, nb, pltpu.async_copy)
            for w in gather_copies(b, pltpu.make_async_copy):
                w.wait()
            write_copies(s, b, pltpu.async_copy)
        return carry

    lax.fori_loop(0, NSTAGE // 2, body, 0)
    for w in write_copies(NSTAGE - 2, 0, pltpu.make_async_copy):
        w.wait()
    for w in write_copies(NSTAGE - 1, 1, pltpu.make_async_copy):
        w.wait()


def kernel(pad_chars, pad_bichars, pad_segs, pad_poss, char_table, bichar_table, seg_table, pos_table):
    def flat(a):
        return a.astype(jnp.int32).reshape(N // CHUNK, CHUNK)

    out = _mix_embed(flat(pad_chars), flat(pad_segs), flat(pad_poss), flat(pad_bichars),
                     char_table, seg_table, pos_table, bichar_table)
    return out.reshape(B, L, D_TOT)


# speedup vs baseline: 2.0230x; 2.0067x over previous
import functools

import jax
import jax.numpy as jnp
from jax import lax
from jax.experimental import pallas as pl
from jax.experimental.pallas import tpu as pltpu
from jax.experimental.pallas import tpu_sc as plsc

B, L = 4096, 200
CHAR_D, SEG_D, POS_D, BICHAR_D = 64, 16, 32, 64
D_TOT = CHAR_D + SEG_D + POS_D + BICHAR_D
N = B * L

NC, NS = 2, 16
NW = NC * NS
TOK_PER_W = N // NW
CHUNK = 128
SUP = 2
STAGE_TOK = SUP * CHUNK
NSTAGE = TOK_PER_W // STAGE_TOK

_mesh = plsc.VectorSubcoreMesh(core_axis_name="c", subcore_axis_name="s")

_COLS = (
    (0, CHAR_D),
    (CHAR_D, SEG_D),
    (CHAR_D + SEG_D, POS_D),
    (CHAR_D + SEG_D + POS_D, BICHAR_D),
)
_DIMS = (CHAR_D, SEG_D, POS_D, BICHAR_D)


@functools.partial(
    pl.kernel,
    mesh=_mesh,
    out_type=jax.ShapeDtypeStruct((N, D_TOT), jnp.float32),
    scratch_types=[
        [pltpu.VMEM((SUP, CHUNK), jnp.int32) for _ in range(4)],
        [pltpu.VMEM((SUP, CHUNK), jnp.int32) for _ in range(4)],
        [pltpu.VMEM((STAGE_TOK, d), jnp.float32) for d in _DIMS],
        [pltpu.VMEM((STAGE_TOK, d), jnp.float32) for d in _DIMS],
        pltpu.VMEM((8, SEG_D), jnp.float32),
        pltpu.VMEM((512, POS_D), jnp.float32),
        pltpu.SemaphoreType.DMA,
        pltpu.SemaphoreType.DMA,
        pltpu.SemaphoreType.DMA,
        pltpu.SemaphoreType.DMA,
        pltpu.SemaphoreType.DMA,
        pltpu.SemaphoreType.DMA,
    ],
    compiler_params=pltpu.CompilerParams(use_tc_tiling_on_sc=False,
                                         needs_layout_passes=False),
)
def _mix_embed(ic_hbm, is_hbm, ip_hbm, ib_hbm,
               char_hbm, seg_hbm, pos_hbm, bichar_hbm, out_hbm,
               idx0, idx1, bufs0, bufs1, seg_t, pos_t,
               is0, is1, gs0, gs1, ws0, ws1):
    idx_v = (idx0, idx1)
    tab_v = (bufs0, bufs1)
    isem = (is0, is1)
    gsem = (gs0, gs1)
    wsem = (ws0, ws1)
    idx_hbm = (ic_hbm, is_hbm, ip_hbm, ib_hbm)
    tables = (char_hbm, None, None, bichar_hbm)
    wid = lax.axis_index("s") * NC + lax.axis_index("c")
    first_stage = wid * NSTAGE

    pltpu.sync_copy(seg_hbm, seg_t)
    pltpu.sync_copy(pos_hbm, pos_t)
    lane = jnp.arange(16, dtype=jnp.int32)

    def idx_copies(s, b, fn):
        row = (first_stage + s) * SUP
        return [fn(idx_hbm[t].at[pl.ds(row, SUP)], idx_v[b][t], isem[b])
                for t in range(4)]

    def gather_copies(b, fn):
        out = []
        for c in range(SUP):
            tok = c * CHUNK
            for t in (0, 3):
                out.append(fn(
                    tables[t].at[idx_v[b][t].at[c]],
                    tab_v[b][t].at[pl.ds(tok, CHUNK)],
                    gsem[b],
                ))
        return out

    def seg_pos_fill(b):
        seg_buf, pos_buf = tab_v[b][1], tab_v[b][2]
        for c in range(SUP):
            def grp_body(g, carry):
                j0 = g * 16
                vseg = idx_v[b][1][c, pl.ds(j0, 16)]
                vpos = idx_v[b][2][c, pl.ds(j0, 16)]
                trow = (c * CHUNK + j0) + lane
                for col in range(SEG_D):
                    colv = jnp.full((16,), col, dtype=jnp.int32)
                    plsc.store_scatter(seg_buf, [trow, colv],
                                       plsc.load_gather(seg_t, [vseg, colv]))
                for col in range(POS_D):
                    colv = jnp.full((16,), col, dtype=jnp.int32)
                    plsc.store_scatter(pos_buf, [trow, colv],
                                       plsc.load_gather(pos_t, [vpos, colv]))
                return carry
            lax.fori_loop(0, CHUNK // 16, grp_body, 0)

    def write_copies(s, b, fn):
        base = (first_stage + s) * STAGE_TOK
        return [
            fn(tab_v[b][t], out_hbm.at[pl.ds(base, STAGE_TOK), pl.ds(col, width)], wsem[b])
            for t, (col, width) in enumerate(_COLS)
        ]

    idx_copies(0, 0, pltpu.async_copy)

    def body(ss, carry):
        for b in (0, 1):
            s = ss * 2 + b
            nb = 1 - b
            for w in idx_copies(s, b, pltpu.make_async_copy):
                w.wait()
            @pl.when(s >= 2)
            def _():
                for w in write_copies(s - 2, b, pltpu.make_async_copy):
                    w.wait()
            gather_copies(b, pltpu.async_copy)
            @pl.when(s + 1 < NSTAGE)
            def _():
                idx_copies(s + 1, nb, pltpu.async_copy)
            seg_pos_fill(b)
            for w in gather_copies(b, pltpu.make_async_copy):
                w.wait()
            write_copies(s, b, pltpu.async_copy)
        return carry

    lax.fori_loop(0, NSTAGE // 2, body, 0)
    for w in write_copies(NSTAGE - 2, 0, pltpu.make_async_copy):
        w.wait()
    for w in write_copies(NSTAGE - 1, 1, pltpu.make_async_copy):
        w.wait()


def kernel(pad_chars, pad_bichars, pad_segs, pad_poss, char_table, bichar_table, seg_table, pos_table):
    def flat(a):
        return a.astype(jnp.int32).reshape(N // CHUNK, CHUNK)

    out = _mix_embed(flat(pad_chars), flat(pad_segs), flat(pad_poss), flat(pad_bichars),
                     char_table, seg_table, pos_table, bichar_table)
    return out.reshape(B, L, D_TOT)
